# baseline (speedup 1.0000x reference)
import functools

import jax
import jax.numpy as jnp
from jax import lax
from jax.experimental import pallas as pl
from jax.experimental.pallas import tpu as pltpu
from jax.experimental.pallas import tpu_sc as plsc

_N = 1024
_D = _N - 1
_E = 65536
_H = 128
_OUT = _N * _D // 2
_ND = _N * _D

_NC = 2
_NS = 16
_EPT = _E // (_NC * _NS)
_IDXR = _EPT // 128
_BSL = (_N * _N) // _NS
_ZCH = 8192


def _count_body(edges, out, row_v, col_v, idx_v, ones_v, zero_v, b_sh):
    c = lax.axis_index("c")
    s = lax.axis_index("s")

    def _zfill(i, carry):
        zero_v[pl.ds(i * 16, 16)] = jnp.zeros((16,), jnp.float32)
        return carry

    lax.fori_loop(0, _ZCH // 16, _zfill, 0)
    for l in range(128 // 16):
        ones_v[pl.ds(l * 16, 16)] = jnp.ones((16,), jnp.float32)

    for j in range(_BSL // _ZCH):
        pltpu.sync_copy(zero_v, b_sh.at[pl.ds(s * _BSL + j * _ZCH, _ZCH)])
    plsc.subcore_barrier()

    g = c * _NS + s
    pltpu.sync_copy(edges.at[0, pl.ds(g * _EPT, _EPT)], row_v)
    pltpu.sync_copy(edges.at[1, pl.ds(g * _EPT, _EPT)], col_v)

    for j in range(_IDXR):
        for l in range(128 // 16):
            e0 = j * 128 + l * 16
            r16 = row_v[pl.ds(e0, 16)]
            c16 = col_v[pl.ds(e0, 16)]
            idx_v[j, pl.ds(l * 16, 16)] = c16 * _N + r16
    for j in range(_IDXR):
        pltpu.sync_copy(ones_v, b_sh.at[idx_v.at[j]], add=True)

    plsc.subcore_barrier()
    pltpu.sync_copy(b_sh.at[pl.ds(s * _BSL, _BSL)],
                    out.at[c, pl.ds(s * _BSL, _BSL)])


@functools.lru_cache(maxsize=1)
def _count_edges_kernel():
    return pl.kernel(
        _count_body,
        out_type=jax.ShapeDtypeStruct((_NC, _N * _N), jnp.float32),
        mesh=plsc.VectorSubcoreMesh(core_axis_name="c", subcore_axis_name="s",
                                    num_cores=_NC, num_subcores=_NS),
        scratch_types=[
            pltpu.VMEM((_EPT,), jnp.int32),
            pltpu.VMEM((_EPT,), jnp.int32),
            pltpu.VMEM((_IDXR, 128), jnp.int32),
            pltpu.VMEM((128,), jnp.float32),
            pltpu.VMEM((_ZCH,), jnp.float32),
            pltpu.VMEM_SHARED((_N * _N,), jnp.float32),
        ],
    )


def _gcn_body(b0_ref, b1_ref, x_ref, w1_ref, v1_ref, w2_ref, v2_ref,
              w3_ref, v3_ref, g3_ref):
    B = b0_ref[...] + b1_ref[...]
    ii = lax.broadcasted_iota(jnp.int32, (_N, _N), 0)
    jj = lax.broadcasted_iota(jnp.int32, (_N, _N), 1)
    eye = jnp.where(ii == jj, jnp.float32(1.0), jnp.float32(0.0))
    ones_col = jnp.ones((_N, 1), jnp.float32)
    ones_row = jnp.ones((1, _N), jnp.float32)
    deg_c = jnp.dot(B, ones_col, preferred_element_type=jnp.float32) + 1.0
    deg_r = lax.dot_general(ones_row, B, (((1,), (1,)), ((), ())),
                            preferred_element_type=jnp.float32) + 1.0
    A = (B + eye) * lax.rsqrt(deg_c) * lax.rsqrt(deg_r)

    def gcn(z, w_ref, v_ref):
        zl = lax.dot_general(z, w_ref[...], (((1,), (1,)), ((), ())),
                             preferred_element_type=jnp.float32)
        return jnp.dot(A, zl, preferred_element_type=jnp.float32) + v_ref[...]

    g1 = gcn(x_ref[...], w1_ref, v1_ref)
    g2 = jnp.maximum(gcn(g1, w2_ref, v2_ref) + g1, 0.0)
    g3 = jnp.maximum(gcn(g2, w3_ref, v3_ref) + g2, 0.0)
    g3_ref[...] = g3


_gcn_layers = pl.pallas_call(
    _gcn_body,
    out_shape=jax.ShapeDtypeStruct((_N, _D), jnp.float32),
)


_M4 = _ND // _H
_MB = 88
_MSC = 4928
_MOFF = _MSC // _MB
_G4 = (_M4 - _MSC) // _MB


_S4 = 4
_JS = _H // _S4
_CB = _MB * _H


def _mv4_body(y_ref, w4a_ref, w4b_ref, w4c_ref, w4d_ref, h_ref, acc_ref):
    k = pl.program_id(0)

    @pl.when(k == 0)
    def _init():
        acc_ref[...] = jnp.zeros_like(acc_ref)

    y = y_ref[...]
    hs = [lax.dot_general(y, wr[...], (((1,), (1,)), ((), ())),
                          preferred_element_type=jnp.float32)
          for wr in (w4a_ref, w4b_ref, w4c_ref, w4d_ref)]
    acc_ref[...] += jnp.concatenate(hs, axis=1)

    @pl.when(k == _G4 - 1)
    def _fin():
        h_ref[...] = acc_ref[...]


def _w4_spec(i):
    return pl.BlockSpec((_JS, _CB), lambda k, i=i: (i, k + _MOFF))


_mv4 = pl.pallas_call(
    _mv4_body,
    grid=(_G4,),
    in_specs=[
        pl.BlockSpec((1, _CB), lambda k: (0, k + _MOFF)),
        _w4_spec(0), _w4_spec(1), _w4_spec(2), _w4_spec(3),
    ],
    out_specs=pl.BlockSpec((1, _H), lambda k: (0, 0)),
    out_shape=jax.ShapeDtypeStruct((1, _H), jnp.float32),
    scratch_shapes=[pltpu.VMEM((1, _H), jnp.float32)],
)


_CSC = _MSC * _H
_CSC2 = _CSC // _NC
_CCH2 = 4096
_NCH2 = _CSC2 // _CCH2
_TPC = _CCH2 // _H


def _mv4sc_body(w4_hbm, y_hbm, out_hbm, wbuf, ybuf, obuf):
    c = lax.axis_index("c")
    s = lax.axis_index("s")
    wid = c * _NS + s

    def chunk(g, accs):
        base = c * _CSC2 + g * _CCH2
        m0 = pl.multiple_of(base // _H, 8)
        pltpu.sync_copy(w4_hbm.at[pl.ds(8 * s, 8), pl.ds(base, _CCH2)], wbuf)
        pltpu.sync_copy(y_hbm.at[pl.ds(m0, _TPC), :], ybuf)

        def tile(t, aa):
            o = t * _H
            new = []
            for r in range(8):
                a = aa[r]
                for v in range(_H // 16):
                    a += (wbuf[r, pl.ds(o + v * 16, 16)]
                          * ybuf[t, pl.ds(v * 16, 16)])
                new.append(a)
            return tuple(new)

        return lax.fori_loop(0, _TPC, tile, accs)

    z = jnp.zeros((16,), jnp.float32)
    accs = lax.fori_loop(0, _NCH2, chunk, (z,) * 8)
    for r in range(8):
        obuf[pl.ds(r * 16, 16)] = accs[r]
    pltpu.sync_copy(obuf, out_hbm.at[pl.ds(wid * 128, 128)])


@functools.lru_cache(maxsize=1)
def _mv4sc_kernel():
    return pl.kernel(
        _mv4sc_body,
        out_type=jax.ShapeDtypeStruct((_NC * _NS * 128,), jnp.float32),
        mesh=plsc.VectorSubcoreMesh(core_axis_name="c", subcore_axis_name="s",
                                    num_cores=_NC, num_subcores=_NS),
        scratch_types=[
            pltpu.VMEM((8, _CCH2), jnp.float32),
            pltpu.VMEM((_TPC, _H), jnp.float32),
            pltpu.VMEM((128,), jnp.float32),
        ],
        compiler_params=pltpu.CompilerParams(use_tc_tiling_on_sc=True),
    )


_G5 = 32
_S5 = 2
_RB = _OUT // (_G5 * _S5)


def _mv5_body(h_ref, w5a_ref, w5b_ref, b5_ref, o_ref):
    h = h_ref[...]
    res_a = lax.dot_general(h, w5a_ref[...], (((1,), (1,)), ((), ())),
                            preferred_element_type=jnp.float32)
    res_b = lax.dot_general(h, w5b_ref[...], (((1,), (1,)), ((), ())),
                            preferred_element_type=jnp.float32)
    res = jnp.concatenate([res_a, res_b], axis=0)
    o_ref[...] = (res + b5_ref[...][0])[None]


_mv5 = pl.pallas_call(
    _mv5_body,
    grid=(_G5,),
    in_specs=[
        pl.BlockSpec((1, _H), lambda k: (0, 0)),
        pl.BlockSpec((_RB, _H), lambda k: (2 * k, 0)),
        pl.BlockSpec((_RB, _H), lambda k: (2 * k + 1, 0)),
        pl.BlockSpec((1, _S5, _RB), lambda k: (k, 0, 0)),
    ],
    out_specs=pl.BlockSpec((1, _S5, _RB), lambda k: (k, 0, 0)),
    out_shape=jax.ShapeDtypeStruct((_G5, _S5, _RB), jnp.float32),
)


@jax.jit
def kernel(x, edge_index, W1, b1, W2, b2, W3, b3, W4, b4, W5, b5):
    parts = _count_edges_kernel()(edge_index)
    g3 = _gcn_layers(parts[0].reshape(_N, _N), parts[1].reshape(_N, _N),
                     x, W1, b1.reshape(1, _D), W2, b2.reshape(1, _D),
                     W3, b3.reshape(1, _D))
    h_sc_parts = _mv4sc_kernel()(W4, g3.reshape(_M4, _H))
    h_tc = _mv4(g3.reshape(1, _ND), W4, W4, W4, W4)
    h_sc = jnp.sum(h_sc_parts.reshape(_NC, _H, 16), axis=(0, 2))
    h = jnp.maximum(h_tc + h_sc.reshape(1, _H) + b4.reshape(1, _H), 0.0)
    out = _mv5(h, W5, W5, b5.reshape(_G5, _S5, _RB))
    return out.reshape(_OUT)


# speedup vs baseline: 1.0986x; 1.0986x over previous
import functools

import jax
import jax.numpy as jnp
from jax import lax
from jax.experimental import pallas as pl
from jax.experimental.pallas import tpu as pltpu
from jax.experimental.pallas import tpu_sc as plsc

_N = 1024
_D = _N - 1
_E = 65536
_H = 128
_OUT = _N * _D // 2
_ND = _N * _D

_NC = 2
_NS = 16
_EPT = _E // (_NC * _NS)
_IDXR = _EPT // 128
_BSL = (_N * _N) // _NS
_ZCH = 8192


def _count_body(edges, out, row_v, col_v, idx_v, ones_v, zero_v, b_sh):
    c = lax.axis_index("c")
    s = lax.axis_index("s")

    def _zfill(i, carry):
        zero_v[pl.ds(i * 16, 16)] = jnp.zeros((16,), jnp.float32)
        return carry

    lax.fori_loop(0, _ZCH // 16, _zfill, 0)
    for l in range(128 // 16):
        ones_v[pl.ds(l * 16, 16)] = jnp.ones((16,), jnp.float32)

    for j in range(_BSL // _ZCH):
        pltpu.sync_copy(zero_v, b_sh.at[pl.ds(s * _BSL + j * _ZCH, _ZCH)])
    plsc.subcore_barrier()

    g = c * _NS + s
    pltpu.sync_copy(edges.at[0, pl.ds(g * _EPT, _EPT)], row_v)
    pltpu.sync_copy(edges.at[1, pl.ds(g * _EPT, _EPT)], col_v)

    for j in range(_IDXR):
        for l in range(128 // 16):
            e0 = j * 128 + l * 16
            r16 = row_v[pl.ds(e0, 16)]
            c16 = col_v[pl.ds(e0, 16)]
            idx_v[j, pl.ds(l * 16, 16)] = c16 * _N + r16
    for j in range(_IDXR):
        pltpu.sync_copy(ones_v, b_sh.at[idx_v.at[j]], add=True)

    plsc.subcore_barrier()
    pltpu.sync_copy(b_sh.at[pl.ds(s * _BSL, _BSL)],
                    out.at[c, pl.ds(s * _BSL, _BSL)])


@functools.lru_cache(maxsize=1)
def _count_edges_kernel():
    return pl.kernel(
        _count_body,
        out_type=jax.ShapeDtypeStruct((_NC, _N * _N), jnp.float32),
        mesh=plsc.VectorSubcoreMesh(core_axis_name="c", subcore_axis_name="s",
                                    num_cores=_NC, num_subcores=_NS),
        scratch_types=[
            pltpu.VMEM((_EPT,), jnp.int32),
            pltpu.VMEM((_EPT,), jnp.int32),
            pltpu.VMEM((_IDXR, 128), jnp.int32),
            pltpu.VMEM((128,), jnp.float32),
            pltpu.VMEM((_ZCH,), jnp.float32),
            pltpu.VMEM_SHARED((_N * _N,), jnp.float32),
        ],
    )


def _gcn_body(b0_ref, b1_ref, x_ref, w1_ref, v1_ref, w2_ref, v2_ref,
              w3_ref, v3_ref, g3_ref):
    B = b0_ref[...] + b1_ref[...]
    ii = lax.broadcasted_iota(jnp.int32, (_N, _N), 0)
    jj = lax.broadcasted_iota(jnp.int32, (_N, _N), 1)
    eye = jnp.where(ii == jj, jnp.float32(1.0), jnp.float32(0.0))
    ones_col = jnp.ones((_N, 1), jnp.float32)
    ones_row = jnp.ones((1, _N), jnp.float32)
    deg_c = jnp.dot(B, ones_col, preferred_element_type=jnp.float32) + 1.0
    deg_r = lax.dot_general(ones_row, B, (((1,), (1,)), ((), ())),
                            preferred_element_type=jnp.float32) + 1.0
    A = (B + eye) * lax.rsqrt(deg_c) * lax.rsqrt(deg_r)

    def gcn(z, w_ref, v_ref):
        zl = lax.dot_general(z, w_ref[...], (((1,), (1,)), ((), ())),
                             preferred_element_type=jnp.float32)
        return jnp.dot(A, zl, preferred_element_type=jnp.float32) + v_ref[...]

    g1 = gcn(x_ref[...], w1_ref, v1_ref)
    g2 = jnp.maximum(gcn(g1, w2_ref, v2_ref) + g1, 0.0)
    g3 = jnp.maximum(gcn(g2, w3_ref, v3_ref) + g2, 0.0)
    g3_ref[...] = g3


_gcn_layers = pl.pallas_call(
    _gcn_body,
    out_shape=jax.ShapeDtypeStruct((_N, _D), jnp.float32),
)


_M4 = _ND // _H
_MSC = 4992
_MTC = _M4 - _MSC
_MB = 168
_G4 = _MTC // _MB


_S4 = 4
_JS = _H // _S4
_CB = _MB * _H


def _mv4_body(y_ref, w4a_ref, w4b_ref, w4c_ref, w4d_ref, h_ref, acc_ref):
    k = pl.program_id(0)

    @pl.when(k == 0)
    def _init():
        acc_ref[...] = jnp.zeros_like(acc_ref)

    y = y_ref[...]
    hs = [lax.dot_general(y, wr[...], (((1,), (1,)), ((), ())),
                          preferred_element_type=jnp.float32)
          for wr in (w4a_ref, w4b_ref, w4c_ref, w4d_ref)]
    acc_ref[...] += jnp.concatenate(hs, axis=1)

    @pl.when(k == _G4 - 1)
    def _fin():
        h_ref[...] = acc_ref[...]


def _w4_spec(i):
    return pl.BlockSpec((_JS, _CB), lambda k, i=i: (i, k))


_mv4 = pl.pallas_call(
    _mv4_body,
    grid=(_G4,),
    in_specs=[
        pl.BlockSpec((1, _CB), lambda k: (0, k)),
        _w4_spec(0), _w4_spec(1), _w4_spec(2), _w4_spec(3),
    ],
    out_specs=pl.BlockSpec((1, _H), lambda k: (0, 0)),
    out_shape=jax.ShapeDtypeStruct((1, _H), jnp.float32),
    scratch_shapes=[pltpu.VMEM((1, _H), jnp.float32)],
)


_COFF = _MTC * _H
_CSC = _MSC * _H
_CSC2 = _CSC // _NC
_CCH2 = 8192
_NCH2 = _CSC2 // _CCH2
_TPC = _CCH2 // _H


def _mv4sc_body(w4_hbm, y_hbm, out_hbm, wbuf, ybuf, obuf):
    c = lax.axis_index("c")
    s = lax.axis_index("s")
    wid = c * _NS + s

    def chunk(g, accs):
        base = _COFF + c * _CSC2 + g * _CCH2
        m0 = pl.multiple_of(base // _H, 8)
        pltpu.sync_copy(w4_hbm.at[pl.ds(8 * s, 8), pl.ds(base, _CCH2)], wbuf)
        pltpu.sync_copy(y_hbm.at[pl.ds(m0, _TPC), :], ybuf)

        def tile(t, aa):
            o = t * _H
            new = []
            for r in range(8):
                a = aa[r]
                for v in range(_H // 16):
                    a += (wbuf[r, pl.ds(o + v * 16, 16)]
                          * ybuf[t, pl.ds(v * 16, 16)])
                new.append(a)
            return tuple(new)

        return lax.fori_loop(0, _TPC, tile, accs)

    z = jnp.zeros((16,), jnp.float32)
    accs = lax.fori_loop(0, _NCH2, chunk, (z,) * 8)
    for r in range(8):
        obuf[pl.ds(r * 16, 16)] = accs[r]
    pltpu.sync_copy(obuf, out_hbm.at[pl.ds(wid * 128, 128)])


@functools.lru_cache(maxsize=1)
def _mv4sc_kernel():
    return pl.kernel(
        _mv4sc_body,
        out_type=jax.ShapeDtypeStruct((_NC * _NS * 128,), jnp.float32),
        mesh=plsc.VectorSubcoreMesh(core_axis_name="c", subcore_axis_name="s",
                                    num_cores=_NC, num_subcores=_NS),
        scratch_types=[
            pltpu.VMEM((8, _CCH2), jnp.float32),
            pltpu.VMEM((_TPC, _H), jnp.float32),
            pltpu.VMEM((128,), jnp.float32),
        ],
        compiler_params=pltpu.CompilerParams(use_tc_tiling_on_sc=True),
    )


_G5 = 32
_S5 = 2
_RB = _OUT // (_G5 * _S5)


def _mv5_body(h_ref, w5a_ref, w5b_ref, b5_ref, o_ref):
    h = h_ref[...]
    res_a = lax.dot_general(h, w5a_ref[...], (((1,), (1,)), ((), ())),
                            preferred_element_type=jnp.float32)
    res_b = lax.dot_general(h, w5b_ref[...], (((1,), (1,)), ((), ())),
                            preferred_element_type=jnp.float32)
    res = jnp.concatenate([res_a, res_b], axis=0)
    o_ref[...] = (res + b5_ref[...][0])[None]


_mv5 = pl.pallas_call(
    _mv5_body,
    grid=(_G5,),
    in_specs=[
        pl.BlockSpec((1, _H), lambda k: (0, 0)),
        pl.BlockSpec((_RB, _H), lambda k: (2 * k, 0)),
        pl.BlockSpec((_RB, _H), lambda k: (2 * k + 1, 0)),
        pl.BlockSpec((1, _S5, _RB), lambda k: (k, 0, 0)),
    ],
    out_specs=pl.BlockSpec((1, _S5, _RB), lambda k: (k, 0, 0)),
    out_shape=jax.ShapeDtypeStruct((_G5, _S5, _RB), jnp.float32),
)


@jax.jit
def kernel(x, edge_index, W1, b1, W2, b2, W3, b3, W4, b4, W5, b5):
    parts = _count_edges_kernel()(edge_index)
    g3 = _gcn_layers(parts[0].reshape(_N, _N), parts[1].reshape(_N, _N),
                     x, W1, b1.reshape(1, _D), W2, b2.reshape(1, _D),
                     W3, b3.reshape(1, _D))
    h_sc_parts = _mv4sc_kernel()(W4, g3.reshape(_M4, _H))
    h_tc = _mv4(g3.reshape(1, _ND), W4, W4, W4, W4)
    h_sc = jnp.sum(h_sc_parts.reshape(_NC, _H, 16), axis=(0, 2))
    h = jnp.maximum(h_tc + h_sc.reshape(1, _H) + b4.reshape(1, _H), 0.0)
    out = _mv5(h, W5, W5, b5.reshape(_G5, _S5, _RB))
    return out.reshape(_OUT)


# speedup vs baseline: 1.4064x; 1.2802x over previous
import functools

import jax
import jax.numpy as jnp
from jax import lax
from jax.experimental import pallas as pl
from jax.experimental.pallas import tpu as pltpu
from jax.experimental.pallas import tpu_sc as plsc

_N = 1024
_D = _N - 1
_E = 65536
_H = 128
_OUT = _N * _D // 2
_ND = _N * _D

_NC = 2
_NS = 16
_EPT = _E // (_NC * _NS)
_IDXR = _EPT // 128
_BSL = (_N * _N) // _NS
_ZCH = 8192


def _count_body(edges, out, row_v, col_v, idx_v, ones_v, zero_v, b_sh):
    c = lax.axis_index("c")
    s = lax.axis_index("s")

    def _zfill(i, carry):
        zero_v[pl.ds(i * 16, 16)] = jnp.zeros((16,), jnp.float32)
        return carry

    lax.fori_loop(0, _ZCH // 16, _zfill, 0)
    for l in range(128 // 16):
        ones_v[pl.ds(l * 16, 16)] = jnp.ones((16,), jnp.float32)

    for j in range(_BSL // _ZCH):
        pltpu.sync_copy(zero_v, b_sh.at[pl.ds(s * _BSL + j * _ZCH, _ZCH)])
    plsc.subcore_barrier()

    g = c * _NS + s
    pltpu.sync_copy(edges.at[0, pl.ds(g * _EPT, _EPT)], row_v)
    pltpu.sync_copy(edges.at[1, pl.ds(g * _EPT, _EPT)], col_v)

    for j in range(_IDXR):
        for l in range(128 // 16):
            e0 = j * 128 + l * 16
            r16 = row_v[pl.ds(e0, 16)]
            c16 = col_v[pl.ds(e0, 16)]
            idx_v[j, pl.ds(l * 16, 16)] = c16 * _N + r16
    for j in range(_IDXR):
        pltpu.sync_copy(ones_v, b_sh.at[idx_v.at[j]], add=True)

    plsc.subcore_barrier()
    pltpu.sync_copy(b_sh.at[pl.ds(s * _BSL, _BSL)],
                    out.at[c, pl.ds(s * _BSL, _BSL)])


@functools.lru_cache(maxsize=1)
def _count_edges_kernel():
    return pl.kernel(
        _count_body,
        out_type=jax.ShapeDtypeStruct((_NC, _N * _N), jnp.float32),
        mesh=plsc.VectorSubcoreMesh(core_axis_name="c", subcore_axis_name="s",
                                    num_cores=_NC, num_subcores=_NS),
        scratch_types=[
            pltpu.VMEM((_EPT,), jnp.int32),
            pltpu.VMEM((_EPT,), jnp.int32),
            pltpu.VMEM((_IDXR, 128), jnp.int32),
            pltpu.VMEM((128,), jnp.float32),
            pltpu.VMEM((_ZCH,), jnp.float32),
            pltpu.VMEM_SHARED((_N * _N,), jnp.float32),
        ],
    )


def _gcn_body(b0_ref, b1_ref, x_ref, w1_ref, v1_ref, w2_ref, v2_ref,
              w3_ref, v3_ref, g3_ref):
    B = b0_ref[...] + b1_ref[...]
    ii = lax.broadcasted_iota(jnp.int32, (_N, _N), 0)
    jj = lax.broadcasted_iota(jnp.int32, (_N, _N), 1)
    eye = jnp.where(ii == jj, jnp.float32(1.0), jnp.float32(0.0))
    ones_col = jnp.ones((_N, 1), jnp.float32)
    ones_row = jnp.ones((1, _N), jnp.float32)
    deg_c = jnp.dot(B, ones_col, preferred_element_type=jnp.float32) + 1.0
    deg_r = lax.dot_general(ones_row, B, (((1,), (1,)), ((), ())),
                            preferred_element_type=jnp.float32) + 1.0
    A = (B + eye) * lax.rsqrt(deg_c) * lax.rsqrt(deg_r)

    def gcn(z, w_ref, v_ref):
        zl = lax.dot_general(z, w_ref[...], (((1,), (1,)), ((), ())),
                             preferred_element_type=jnp.float32)
        return jnp.dot(A, zl, preferred_element_type=jnp.float32) + v_ref[...]

    g1 = gcn(x_ref[...], w1_ref, v1_ref)
    g2 = jnp.maximum(gcn(g1, w2_ref, v2_ref) + g1, 0.0)
    g3 = jnp.maximum(gcn(g2, w3_ref, v3_ref) + g2, 0.0)
    g3_ref[...] = g3


_gcn_layers = pl.pallas_call(
    _gcn_body,
    out_shape=jax.ShapeDtypeStruct((_N, _D), jnp.float32),
)


_M4 = _ND // _H
_MSC = 2560
_MTC = _M4 - _MSC
_MB = 152
_G4 = _MTC // _MB


_S4 = 4
_JS = _H // _S4
_CB = _MB * _H


def _mv4_body(y_ref, w4a_ref, w4b_ref, w4c_ref, w4d_ref, h_ref, acc_ref):
    k = pl.program_id(0)

    @pl.when(k == 0)
    def _init():
        acc_ref[...] = jnp.zeros_like(acc_ref)

    y = y_ref[...]
    hs = [lax.dot_general(y, wr[...], (((1,), (1,)), ((), ())),
                          preferred_element_type=jnp.float32)
          for wr in (w4a_ref, w4b_ref, w4c_ref, w4d_ref)]
    acc_ref[...] += jnp.concatenate(hs, axis=1)

    @pl.when(k == _G4 - 1)
    def _fin():
        h_ref[...] = acc_ref[...]


def _w4_spec(i):
    return pl.BlockSpec((_JS, _CB), lambda k, i=i: (i, k))


_mv4 = pl.pallas_call(
    _mv4_body,
    grid=(_G4,),
    in_specs=[
        pl.BlockSpec((1, _CB), lambda k: (0, k)),
        _w4_spec(0), _w4_spec(1), _w4_spec(2), _w4_spec(3),
    ],
    out_specs=pl.BlockSpec((1, _H), lambda k: (0, 0)),
    out_shape=jax.ShapeDtypeStruct((1, _H), jnp.float32),
    scratch_shapes=[pltpu.VMEM((1, _H), jnp.float32)],
)


_COFF = _MTC * _H
_CSC = _MSC * _H
_CSC2 = _CSC // _NC
_CCH2 = 8192
_NCH2 = _CSC2 // _CCH2
_TPC = _CCH2 // _H


def _mv4sc_body(w4_hbm, y_hbm, out_hbm, wbuf, ybuf, obuf):
    c = lax.axis_index("c")
    s = lax.axis_index("s")
    wid = c * _NS + s

    def chunk(g, accs):
        base = _COFF + c * _CSC2 + g * _CCH2
        m0 = pl.multiple_of(base // _H, 8)
        pltpu.sync_copy(w4_hbm.at[pl.ds(8 * s, 8), pl.ds(base, _CCH2)], wbuf)
        pltpu.sync_copy(y_hbm.at[pl.ds(m0, _TPC), :], ybuf)

        def tile(t, aa):
            o = t * _H
            new = []
            for r in range(8):
                a = aa[r]
                for v in range(_H // 16):
                    a += (wbuf[r, pl.ds(o + v * 16, 16)]
                          * ybuf[t, pl.ds(v * 16, 16)])
                new.append(a)
            return tuple(new)

        return lax.fori_loop(0, _TPC, tile, accs)

    z = jnp.zeros((16,), jnp.float32)
    accs = lax.fori_loop(0, _NCH2, chunk, (z,) * 8)
    for r in range(8):
        obuf[pl.ds(r * 16, 16)] = accs[r]
    pltpu.sync_copy(obuf, out_hbm.at[pl.ds(wid * 128, 128)])


@functools.lru_cache(maxsize=1)
def _mv4sc_kernel():
    return pl.kernel(
        _mv4sc_body,
        out_type=jax.ShapeDtypeStruct((_NC * _NS * 128,), jnp.float32),
        mesh=plsc.VectorSubcoreMesh(core_axis_name="c", subcore_axis_name="s",
                                    num_cores=_NC, num_subcores=_NS),
        scratch_types=[
            pltpu.VMEM((8, _CCH2), jnp.float32),
            pltpu.VMEM((_TPC, _H), jnp.float32),
            pltpu.VMEM((128,), jnp.float32),
        ],
        compiler_params=pltpu.CompilerParams(use_tc_tiling_on_sc=True),
    )


_G5 = 16
_S5 = 4
_RB = _OUT // (_G5 * _S5)


def _mv5_body(h_ref, w5a_ref, w5b_ref, w5c_ref, w5d_ref, b5_ref, o_ref):
    h = h_ref[...]
    res = jnp.concatenate(
        [lax.dot_general(h, wr[...], (((1,), (1,)), ((), ())),
                         preferred_element_type=jnp.float32)
         for wr in (w5a_ref, w5b_ref, w5c_ref, w5d_ref)], axis=0)
    o_ref[...] = (res + b5_ref[...][0])[None]


def _w5_spec(i):
    return pl.BlockSpec((_RB, _H), lambda k, i=i: (_S5 * k + i, 0))


_mv5 = pl.pallas_call(
    _mv5_body,
    grid=(_G5,),
    in_specs=[
        pl.BlockSpec((1, _H), lambda k: (0, 0)),
        _w5_spec(0), _w5_spec(1), _w5_spec(2), _w5_spec(3),
        pl.BlockSpec((1, _S5, _RB), lambda k: (k, 0, 0)),
    ],
    out_specs=pl.BlockSpec((1, _S5, _RB), lambda k: (k, 0, 0)),
    out_shape=jax.ShapeDtypeStruct((_G5, _S5, _RB), jnp.float32),
)


@jax.jit
def kernel(x, edge_index, W1, b1, W2, b2, W3, b3, W4, b4, W5, b5):
    parts = _count_edges_kernel()(edge_index)
    g3 = _gcn_layers(parts[0].reshape(_N, _N), parts[1].reshape(_N, _N),
                     x, W1, b1.reshape(1, _D), W2, b2.reshape(1, _D),
                     W3, b3.reshape(1, _D))
    h_sc_parts = _mv4sc_kernel()(W4, g3.reshape(_M4, _H))
    h_tc = _mv4(g3.reshape(1, _ND), W4, W4, W4, W4)
    h_sc = jnp.sum(h_sc_parts.reshape(_NC, _H, 16), axis=(0, 2))
    h = jnp.maximum(h_tc + h_sc.reshape(1, _H) + b4.reshape(1, _H), 0.0)
    out = _mv5(h, W5, W5, W5, W5, b5.reshape(_G5, _S5, _RB))
    return out.reshape(_OUT)
